# Initial kernel scaffold; baseline (speedup 1.0000x reference)
import functools

import jax
import jax.numpy as jnp
from jax import lax
from jax.experimental import pallas as pl
from jax.experimental.pallas import tpu as pltpu
from jax.experimental.pallas import tpu_sc as plsc

NC = 2
NS = 16
NW = NC * NS

_f32 = jnp.float32

_MESH = plsc.VectorSubcoreMesh(
    core_axis_name="c", subcore_axis_name="s", num_cores=NC, num_subcores=NS
)


def _deg_call(n, nch, c, rps):
    @functools.partial(
        pl.kernel,
        out_type=jax.ShapeDtypeStruct((NC, n, 16), _f32),
        mesh=_MESH,
        scratch_types=[
            pltpu.VMEM_SHARED((n, 16), _f32),
            pltpu.VMEM((nch, c), jnp.int32),
            pltpu.VMEM((c, 16), _f32),
        ],
    )
    def deg_kernel(dst_hbm, ones_hbm, z16_hbm, out_hbm, acc, didx, ones_v):
        cid = lax.axis_index("c")
        sid = lax.axis_index("s")
        w = cid * NS + sid
        pltpu.sync_copy(z16_hbm, acc.at[pl.ds(sid * rps, rps)])
        pltpu.sync_copy(ones_hbm, ones_v)
        pltpu.sync_copy(dst_hbm.at[w], didx)
        plsc.subcore_barrier()

        @pl.loop(0, nch)
        def _chunk(i):
            pltpu.sync_copy(ones_v, acc.at[didx.at[i]], add=True)

        plsc.subcore_barrier()
        pltpu.sync_copy(
            acc.at[pl.ds(sid * rps, rps)],
            out_hbm.at[cid, pl.ds(sid * rps, rps)],
        )

    return deg_kernel


def _hop_call(n, d, nch, c, rps):
    @functools.partial(
        pl.kernel,
        out_type=jax.ShapeDtypeStruct((NC, n, d), _f32),
        mesh=_MESH,
        scratch_types=[
            pltpu.VMEM_SHARED((n, d), _f32),
            pltpu.VMEM((nch, c), jnp.int32),
            pltpu.VMEM((nch, c), jnp.int32),
            pltpu.VMEM((c, d), _f32),
            pltpu.SemaphoreType.DMA,
        ],
    )
    def hop_kernel(g_hbm, src_hbm, dst_hbm, zrows_hbm, out_hbm,
                   acc, sidx, didx, rows, gsem):
        cid = lax.axis_index("c")
        sid = lax.axis_index("s")
        w = cid * NS + sid
        pltpu.sync_copy(zrows_hbm, acc.at[pl.ds(sid * rps, rps)])
        pltpu.sync_copy(src_hbm.at[w], sidx)
        pltpu.sync_copy(dst_hbm.at[w], didx)
        plsc.subcore_barrier()

        @pl.loop(0, nch)
        def _chunk(i):
            pltpu.async_copy(g_hbm.at[sidx.at[i]], rows, gsem).wait()
            pltpu.sync_copy(rows, acc.at[didx.at[i]], add=True)

        plsc.subcore_barrier()
        pltpu.sync_copy(
            acc.at[pl.ds(sid * rps, rps)],
            out_hbm.at[cid, pl.ds(sid * rps, rps)],
        )

    return hop_kernel


def _lin_relu_scale(x, W11, b11, dp, r):
    n, d = x.shape
    h = W11.shape[1]

    def body(x_ref, w_ref, b_ref, dp_ref, g0_ref, dinv_ref):
        hh = jnp.dot(x_ref[...], w_ref[...], preferred_element_type=_f32)
        hh = jnp.maximum(hh + b_ref[...], 0.0)
        deg = 1.0 + dp_ref[0, :, 0:1] + dp_ref[1, :, 0:1]
        dinv = lax.rsqrt(deg)
        g0_ref[...] = hh * dinv
        dinv_ref[...] = jnp.broadcast_to(dinv, (r, 16))

    return pl.pallas_call(
        body,
        grid=(n // r,),
        in_specs=[
            pl.BlockSpec((r, d), lambda i: (i, 0)),
            pl.BlockSpec((d, h), lambda i: (0, 0)),
            pl.BlockSpec((1, h), lambda i: (0, 0)),
            pl.BlockSpec((NC, r, 16), lambda i: (0, i, 0)),
        ],
        out_specs=[
            pl.BlockSpec((r, h), lambda i: (i, 0)),
            pl.BlockSpec((r, 16), lambda i: (i, 0)),
        ],
        out_shape=[
            jax.ShapeDtypeStruct((n, h), _f32),
            jax.ShapeDtypeStruct((n, 16), _f32),
        ],
    )(x, W11, b11.reshape(1, h), dp)


def _combine(p, g, dinv16, r):
    n, h = g.shape

    def body(p_ref, g_ref, dinv_ref, o_ref):
        dinv = dinv_ref[:, 0:1]
        o_ref[...] = (p_ref[0] + p_ref[1] + g_ref[...]) * (dinv * dinv)

    return pl.pallas_call(
        body,
        grid=(n // r,),
        in_specs=[
            pl.BlockSpec((NC, r, h), lambda i: (0, i, 0)),
            pl.BlockSpec((r, h), lambda i: (i, 0)),
            pl.BlockSpec((r, 16), lambda i: (i, 0)),
        ],
        out_specs=pl.BlockSpec((r, h), lambda i: (i, 0)),
        out_shape=jax.ShapeDtypeStruct((n, h), _f32),
    )(p, g, dinv16)


def _final(p, g, dinv16, W1, b1, r):
    n, h = g.shape
    cls = W1.shape[1]

    def body(p_ref, g_ref, dinv_ref, w_ref, b_ref, o_ref):
        dinv = dinv_ref[:, 0:1]
        h2 = (p_ref[0] + p_ref[1] + g_ref[...]) * dinv
        z = jnp.dot(h2, w_ref[...], preferred_element_type=_f32) + b_ref[...]
        m = jnp.max(z, axis=1, keepdims=True)
        zs = z - m
        lse = jnp.log(jnp.sum(jnp.exp(zs), axis=1, keepdims=True))
        o_ref[...] = zs - lse

    return pl.pallas_call(
        body,
        grid=(n // r,),
        in_specs=[
            pl.BlockSpec((NC, r, h), lambda i: (0, i, 0)),
            pl.BlockSpec((r, h), lambda i: (i, 0)),
            pl.BlockSpec((r, 16), lambda i: (i, 0)),
            pl.BlockSpec((h, cls), lambda i: (0, 0)),
            pl.BlockSpec((1, cls), lambda i: (0, 0)),
        ],
        out_specs=pl.BlockSpec((r, cls), lambda i: (i, 0)),
        out_shape=jax.ShapeDtypeStruct((n, cls), _f32),
    )(p, g, dinv16, W1, b1.reshape(1, cls))


def kernel(x, edge_index, W11, b11, W1, b1):
    n, d = x.shape
    h = W11.shape[1]
    e = edge_index.shape[1]

    c = 80
    per_w = e // NW
    nch = per_w // c
    rps = n // NS
    r = 1000

    src3 = edge_index[0].reshape(NW, nch, c)
    dst3 = edge_index[1].reshape(NW, nch, c)
    ones16 = jnp.ones((c, 16), _f32)
    z16 = jnp.zeros((rps, 16), _f32)
    zrows = jnp.zeros((rps, d), _f32)

    hop = _hop_call(n, h, nch, c, rps)

    dp = _deg_call(n, nch, c, rps)(dst3, ones16, z16)
    g0, dinv16 = _lin_relu_scale(x, W11, b11, dp, r)
    p1 = hop(g0, src3, dst3, zrows)
    g1 = _combine(p1, g0, dinv16, r)
    p2 = hop(g1, src3, dst3, zrows)
    return _final(p2, g1, dinv16, W1, b1, r)


# speedup vs baseline: 15.8441x; 15.8441x over previous
import functools

import jax
import jax.numpy as jnp
from jax import lax
from jax.experimental import pallas as pl
from jax.experimental.pallas import tpu as pltpu
from jax.experimental.pallas import tpu_sc as plsc

NC = 2
NS = 16
NW = NC * NS

_f32 = jnp.float32

_MESH = plsc.VectorSubcoreMesh(
    core_axis_name="c", subcore_axis_name="s", num_cores=NC, num_subcores=NS
)


def _deg_call(npad, nch, c, rps):
    @functools.partial(
        pl.kernel,
        out_type=jax.ShapeDtypeStruct((NC, npad, 16), _f32),
        mesh=_MESH,
        compiler_params=pltpu.CompilerParams(use_tc_tiling_on_sc=False),
        scratch_types=[
            pltpu.VMEM_SHARED((npad, 16), _f32),
            pltpu.VMEM((nch, c), jnp.int32),
            pltpu.VMEM((c, 16), _f32),
        ],
    )
    def deg_kernel(dst_hbm, ones_hbm, z16_hbm, out_hbm, acc, didx, ones_v):
        cid = lax.axis_index("c")
        sid = lax.axis_index("s")
        w = cid * NS + sid
        pltpu.sync_copy(z16_hbm, acc.at[pl.ds(sid * rps, rps)])
        pltpu.sync_copy(ones_hbm, ones_v)
        pltpu.sync_copy(dst_hbm.at[w], didx)
        plsc.subcore_barrier()

        @pl.loop(0, nch)
        def _chunk(i):
            pltpu.sync_copy(ones_v, acc.at[didx.at[i]], add=True)

        plsc.subcore_barrier()
        pltpu.sync_copy(
            acc.at[pl.ds(sid * rps, rps)],
            out_hbm.at[cid, pl.ds(sid * rps, rps)],
        )

    return deg_kernel


def _hop_call(npad, d, nch, c, rps):
    @functools.partial(
        pl.kernel,
        out_type=jax.ShapeDtypeStruct((NC, npad, d), _f32),
        mesh=_MESH,
        scratch_types=[
            pltpu.VMEM_SHARED((npad, d), _f32),
            pltpu.VMEM((nch, c), jnp.int32),
            pltpu.VMEM((nch, c), jnp.int32),
            pltpu.VMEM((c, d), _f32),
            pltpu.SemaphoreType.DMA,
        ],
    )
    def hop_kernel(g_hbm, src_hbm, dst_hbm, zrows_hbm, out_hbm,
                   acc, sidx, didx, rows, gsem):
        cid = lax.axis_index("c")
        sid = lax.axis_index("s")
        w = cid * NS + sid
        pltpu.sync_copy(zrows_hbm, acc.at[pl.ds(sid * rps, rps)])
        pltpu.sync_copy(src_hbm.at[w], sidx)
        pltpu.sync_copy(dst_hbm.at[w], didx)
        plsc.subcore_barrier()

        @pl.loop(0, nch)
        def _chunk(i):
            pltpu.async_copy(g_hbm.at[sidx.at[i]], rows, gsem).wait()
            pltpu.sync_copy(rows, acc.at[didx.at[i]], add=True)

        plsc.subcore_barrier()
        pltpu.sync_copy(
            acc.at[pl.ds(sid * rps, rps)],
            out_hbm.at[cid, pl.ds(sid * rps, rps)],
        )

    return hop_kernel


def _lin_relu_scale(x, W11, b11, dp, r):
    n, d = x.shape
    h = W11.shape[1]

    def body(x_ref, w_ref, b_ref, dp_ref, g0_ref, dinv_ref):
        hh = jnp.dot(x_ref[...], w_ref[...], preferred_element_type=_f32)
        hh = jnp.maximum(hh + b_ref[...], 0.0)
        deg = 1.0 + dp_ref[0, :, 0:1] + dp_ref[1, :, 0:1]
        dinv = lax.rsqrt(deg)
        g0_ref[...] = hh * dinv
        dinv_ref[...] = jnp.broadcast_to(dinv, (r, 16))

    return pl.pallas_call(
        body,
        grid=(n // r,),
        in_specs=[
            pl.BlockSpec((r, d), lambda i: (i, 0)),
            pl.BlockSpec((d, h), lambda i: (0, 0)),
            pl.BlockSpec((1, h), lambda i: (0, 0)),
            pl.BlockSpec((NC, r, 16), lambda i: (0, i, 0)),
        ],
        out_specs=[
            pl.BlockSpec((r, h), lambda i: (i, 0)),
            pl.BlockSpec((r, 16), lambda i: (i, 0)),
        ],
        out_shape=[
            jax.ShapeDtypeStruct((n, h), _f32),
            jax.ShapeDtypeStruct((n, 16), _f32),
        ],
    )(x, W11, b11.reshape(1, h), dp)


def _combine(p, g, dinv16, r):
    n, h = g.shape

    def body(p_ref, g_ref, dinv_ref, o_ref):
        dinv = dinv_ref[:, 0:1]
        o_ref[...] = (p_ref[0] + p_ref[1] + g_ref[...]) * (dinv * dinv)

    return pl.pallas_call(
        body,
        grid=(n // r,),
        in_specs=[
            pl.BlockSpec((NC, r, h), lambda i: (0, i, 0)),
            pl.BlockSpec((r, h), lambda i: (i, 0)),
            pl.BlockSpec((r, 16), lambda i: (i, 0)),
        ],
        out_specs=pl.BlockSpec((r, h), lambda i: (i, 0)),
        out_shape=jax.ShapeDtypeStruct((n, h), _f32),
    )(p, g, dinv16)


def _final(p, g, dinv16, W1, b1, r):
    n, h = g.shape
    cls = W1.shape[1]

    def body(p_ref, g_ref, dinv_ref, w_ref, b_ref, o_ref):
        dinv = dinv_ref[:, 0:1]
        h2 = (p_ref[0] + p_ref[1] + g_ref[...]) * dinv
        z = jnp.dot(h2, w_ref[...], preferred_element_type=_f32) + b_ref[...]
        m = jnp.max(z, axis=1, keepdims=True)
        zs = z - m
        lse = jnp.log(jnp.sum(jnp.exp(zs), axis=1, keepdims=True))
        o_ref[...] = zs - lse

    return pl.pallas_call(
        body,
        grid=(n // r,),
        in_specs=[
            pl.BlockSpec((NC, r, h), lambda i: (0, i, 0)),
            pl.BlockSpec((r, h), lambda i: (i, 0)),
            pl.BlockSpec((r, 16), lambda i: (i, 0)),
            pl.BlockSpec((h, cls), lambda i: (0, 0)),
            pl.BlockSpec((1, cls), lambda i: (0, 0)),
        ],
        out_specs=pl.BlockSpec((r, cls), lambda i: (i, 0)),
        out_shape=jax.ShapeDtypeStruct((n, cls), _f32),
    )(p, g, dinv16, W1, b1.reshape(1, cls))


def kernel(x, edge_index, W11, b11, W1, b1):
    n, d = x.shape
    h = W11.shape[1]
    e = edge_index.shape[1]

    c = 80
    per_w = e // NW
    nch = per_w // c
    rps = -(-n // (NS * 8)) * 8
    npad = rps * NS
    r = 1000

    src3 = edge_index[0].reshape(NW, nch, c)
    dst3 = edge_index[1].reshape(NW, nch, c)
    ones16 = jnp.ones((c, 16), _f32)
    z16 = jnp.zeros((rps, 16), _f32)
    zrows = jnp.zeros((rps, d), _f32)

    hop = _hop_call(npad, h, nch, c, rps)

    dp = _deg_call(npad, nch, c, rps)(dst3, ones16, z16)
    g0, dinv16 = _lin_relu_scale(x, W11, b11, dp, r)
    p1 = hop(g0, src3, dst3, zrows)
    g1 = _combine(p1, g0, dinv16, r)
    p2 = hop(g1, src3, dst3, zrows)
    return _final(p2, g1, dinv16, W1, b1, r)


# speedup vs baseline: 22.6490x; 1.4295x over previous
import functools

import jax
import jax.numpy as jnp
from jax import lax
from jax.experimental import pallas as pl
from jax.experimental.pallas import tpu as pltpu
from jax.experimental.pallas import tpu_sc as plsc

NC = 2
NS = 16
NW = NC * NS

_f32 = jnp.float32

_MESH = plsc.VectorSubcoreMesh(
    core_axis_name="c", subcore_axis_name="s", num_cores=NC, num_subcores=NS
)


def _deg_call(npad, nch, c, rps):
    @functools.partial(
        pl.kernel,
        out_type=jax.ShapeDtypeStruct((NC, npad, 16), _f32),
        mesh=_MESH,
        compiler_params=pltpu.CompilerParams(use_tc_tiling_on_sc=False),
        scratch_types=[
            pltpu.VMEM_SHARED((npad, 16), _f32),
            pltpu.VMEM((nch, c), jnp.int32),
            pltpu.VMEM((c, 16), _f32),
        ],
    )
    def deg_kernel(dst_hbm, ones_hbm, z16_hbm, out_hbm, acc, didx, ones_v):
        cid = lax.axis_index("c")
        sid = lax.axis_index("s")
        w = cid * NS + sid
        pltpu.sync_copy(z16_hbm, acc.at[pl.ds(sid * rps, rps)])
        pltpu.sync_copy(ones_hbm, ones_v)
        pltpu.sync_copy(dst_hbm.at[w], didx)
        plsc.subcore_barrier()

        @pl.loop(0, nch)
        def _chunk(i):
            pltpu.sync_copy(ones_v, acc.at[didx.at[i]], add=True)

        plsc.subcore_barrier()
        pltpu.sync_copy(
            acc.at[pl.ds(sid * rps, rps)],
            out_hbm.at[cid, pl.ds(sid * rps, rps)],
        )

    return deg_kernel


def _hop_call(npad, d, nch, c, rps):
    nblk = 5
    bch = nch // nblk
    bring = bch - (bch % 2)

    @functools.partial(
        pl.kernel,
        out_type=jax.ShapeDtypeStruct((NC, npad, d), _f32),
        mesh=_MESH,
        scratch_types=[
            pltpu.VMEM_SHARED((npad, d), _f32),
            pltpu.VMEM((bch, c), jnp.int32),
            pltpu.VMEM((bch, c), jnp.int32),
            pltpu.VMEM((c, d), _f32),
            pltpu.VMEM((c, d), _f32),
            pltpu.SemaphoreType.DMA,
            pltpu.SemaphoreType.DMA,
        ],
    )
    def hop_kernel(g_hbm, src_hbm, dst_hbm, zrows_hbm, out_hbm,
                   acc, sidx, didx, rows0, rows1, gsem0, gsem1):
        cid = lax.axis_index("c")
        sid = lax.axis_index("s")
        w = cid * NS + sid
        rows = (rows0, rows1)
        gsem = (gsem0, gsem1)

        pltpu.sync_copy(zrows_hbm, acc.at[pl.ds(sid * rps, rps)])
        plsc.subcore_barrier()

        @pl.loop(0, nblk)
        def _blk(j):
            pltpu.sync_copy(src_hbm.at[j * NW + w], sidx)
            pltpu.sync_copy(dst_hbm.at[j * NW + w], didx)
            pend = [
                pltpu.async_copy(g_hbm.at[sidx.at[0]], rows0, gsem0),
                pltpu.async_copy(g_hbm.at[sidx.at[1]], rows1, gsem1),
            ]
            for i in range(bch):
                b = i % 2
                pend[b].wait()
                pltpu.sync_copy(rows[b], acc.at[didx.at[i]], add=True)
                if i + 2 < bch:
                    pend[b] = pltpu.async_copy(
                        g_hbm.at[sidx.at[i + 2]], rows[b], gsem[b]
                    )

        plsc.subcore_barrier()
        pltpu.sync_copy(
            acc.at[pl.ds(sid * rps, rps)],
            out_hbm.at[cid, pl.ds(sid * rps, rps)],
        )

    return hop_kernel


def _lin_relu_scale(x, W11, b11, dp, r):
    n, d = x.shape
    h = W11.shape[1]

    def body(x_ref, w_ref, b_ref, dp_ref, g0_ref, dinv_ref):
        hh = jnp.dot(x_ref[...], w_ref[...], preferred_element_type=_f32)
        hh = jnp.maximum(hh + b_ref[...], 0.0)
        deg = 1.0 + dp_ref[0, :, 0:1] + dp_ref[1, :, 0:1]
        dinv = lax.rsqrt(deg)
        g0_ref[...] = hh * dinv
        dinv_ref[...] = jnp.broadcast_to(dinv, (r, 16))

    return pl.pallas_call(
        body,
        grid=(n // r,),
        in_specs=[
            pl.BlockSpec((r, d), lambda i: (i, 0)),
            pl.BlockSpec((d, h), lambda i: (0, 0)),
            pl.BlockSpec((1, h), lambda i: (0, 0)),
            pl.BlockSpec((NC, r, 16), lambda i: (0, i, 0)),
        ],
        out_specs=[
            pl.BlockSpec((r, h), lambda i: (i, 0)),
            pl.BlockSpec((r, 16), lambda i: (i, 0)),
        ],
        out_shape=[
            jax.ShapeDtypeStruct((n, h), _f32),
            jax.ShapeDtypeStruct((n, 16), _f32),
        ],
    )(x, W11, b11.reshape(1, h), dp)


def _combine(p, g, dinv16, r):
    n, h = g.shape

    def body(p_ref, g_ref, dinv_ref, o_ref):
        dinv = dinv_ref[:, 0:1]
        o_ref[...] = (p_ref[0] + p_ref[1] + g_ref[...]) * (dinv * dinv)

    return pl.pallas_call(
        body,
        grid=(n // r,),
        in_specs=[
            pl.BlockSpec((NC, r, h), lambda i: (0, i, 0)),
            pl.BlockSpec((r, h), lambda i: (i, 0)),
            pl.BlockSpec((r, 16), lambda i: (i, 0)),
        ],
        out_specs=pl.BlockSpec((r, h), lambda i: (i, 0)),
        out_shape=jax.ShapeDtypeStruct((n, h), _f32),
    )(p, g, dinv16)


def _final(p, g, dinv16, W1, b1, r):
    n, h = g.shape
    cls = W1.shape[1]

    def body(p_ref, g_ref, dinv_ref, w_ref, b_ref, o_ref):
        dinv = dinv_ref[:, 0:1]
        h2 = (p_ref[0] + p_ref[1] + g_ref[...]) * dinv
        z = jnp.dot(h2, w_ref[...], preferred_element_type=_f32) + b_ref[...]
        m = jnp.max(z, axis=1, keepdims=True)
        zs = z - m
        lse = jnp.log(jnp.sum(jnp.exp(zs), axis=1, keepdims=True))
        o_ref[...] = zs - lse

    return pl.pallas_call(
        body,
        grid=(n // r,),
        in_specs=[
            pl.BlockSpec((NC, r, h), lambda i: (0, i, 0)),
            pl.BlockSpec((r, h), lambda i: (i, 0)),
            pl.BlockSpec((r, 16), lambda i: (i, 0)),
            pl.BlockSpec((h, cls), lambda i: (0, 0)),
            pl.BlockSpec((1, cls), lambda i: (0, 0)),
        ],
        out_specs=pl.BlockSpec((r, cls), lambda i: (i, 0)),
        out_shape=jax.ShapeDtypeStruct((n, cls), _f32),
    )(p, g, dinv16, W1, b1.reshape(1, cls))


def kernel(x, edge_index, W11, b11, W1, b1):
    n, d = x.shape
    h = W11.shape[1]
    e = edge_index.shape[1]

    c = 80
    per_w = e // NW
    nch = per_w // c
    rps = -(-n // (NS * 8)) * 8
    npad = rps * NS
    r = 1000

    nblk = 5
    bch = nch // nblk
    src4 = (edge_index[0].reshape(NW, nblk, bch * c)
            .transpose(1, 0, 2).reshape(nblk * NW, bch, c))
    dst4 = (edge_index[1].reshape(NW, nblk, bch * c)
            .transpose(1, 0, 2).reshape(nblk * NW, bch, c))
    dst3 = edge_index[1].reshape(NW, nch, c)
    ones16 = jnp.ones((c, 16), _f32)
    z16 = jnp.zeros((rps, 16), _f32)
    zrows = jnp.zeros((rps, d), _f32)

    hop = _hop_call(npad, h, nch, c, rps)

    dp = _deg_call(npad, nch, c, rps)(dst3, ones16, z16)
    g0, dinv16 = _lin_relu_scale(x, W11, b11, dp, r)
    p1 = hop(g0, src4, dst4, zrows)
    g1 = _combine(p1, g0, dinv16, r)
    p2 = hop(g1, src4, dst4, zrows)
    return _final(p2, g1, dinv16, W1, b1, r)
